# Initial kernel scaffold; baseline (speedup 1.0000x reference)
import jax
import jax.numpy as jnp
from jax.experimental import pallas as pl


def kernel(q, k, v):
    raise NotImplementedError("write your pallas kernel here")


# speedup vs baseline: 4.1375x; 4.1375x over previous
import jax
import jax.numpy as jnp
from jax import lax
from jax.experimental import pallas as pl
from jax.experimental.pallas import tpu as pltpu
from jax.experimental.pallas import tpu_sc as plsc

_B, _S, _D = 4, 4160, 1024
_L = 13
_SHIFT = 7
_G = _S // _L
_NJ = 4
_LANES = 16
_NW = 32
_CHUNKS = _B * (_G // _NJ)
_CPW = _CHUNKS // _NW

_NET = (
    (0, 1), (2, 3), (4, 5), (6, 7), (8, 9), (10, 11),
    (0, 2), (1, 3), (4, 6), (5, 7), (8, 10), (9, 11),
    (1, 2), (5, 6), (9, 10),
    (0, 4), (1, 5), (2, 6), (3, 7), (8, 12),
    (2, 4), (3, 5), (10, 12),
    (1, 2), (3, 4), (5, 6), (9, 10), (11, 12),
    (0, 8), (1, 9), (2, 10), (3, 11), (4, 12),
    (4, 8), (5, 9), (6, 10), (7, 11),
    (2, 4), (3, 5), (6, 8), (7, 9), (10, 12),
    (1, 2), (3, 4), (5, 6), (7, 8), (9, 10), (11, 12),
)

_wrap_mid = sorted(
    s for j0 in range(0, _G, _NJ)
    if _S - _NJ < (s := (_L - 1) * _G + j0 + _SHIFT) < _S
)
assert _wrap_mid == [_S - 1], _wrap_mid


def _row_copies(l, base, j0, hbm, buf, to_hbm):
    def cp(src, dst):
        if to_hbm:
            pltpu.sync_copy(dst, src)
        else:
            pltpu.sync_copy(src, dst)

    s = l * _G + j0 + _SHIFT
    if l < _L - 1:
        cp(hbm.at[pl.ds((base + s) * _D, _NJ * _D)],
           buf.at[pl.ds(l * _NJ * _D, _NJ * _D)])
        return

    @pl.when(s <= _S - _NJ)
    def _():
        cp(hbm.at[pl.ds((base + s) * _D, _NJ * _D)],
           buf.at[pl.ds(l * _NJ * _D, _NJ * _D)])

    @pl.when((s > _S - _NJ) & (s < _S))
    def _():
        cp(hbm.at[pl.ds((base + _S - 1) * _D, _D)],
           buf.at[pl.ds(l * _NJ * _D, _D)])
        cp(hbm.at[pl.ds(base * _D, (_NJ - 1) * _D)],
           buf.at[pl.ds(l * _NJ * _D + _D, (_NJ - 1) * _D)])

    @pl.when(s >= _S)
    def _():
        cp(hbm.at[pl.ds((base + s - _S) * _D, _NJ * _D)],
           buf.at[pl.ds(l * _NJ * _D, _NJ * _D)])


def _sc_body(v_hbm, out_hbm, buf):
    wid = lax.axis_index("s") * 2 + lax.axis_index("c")

    def chunk_body(t, carry):
        ci = wid * _CPW + t
        b = ci // (_G // _NJ)
        j0 = (ci % (_G // _NJ)) * _NJ
        base = b * _S

        for l in range(_L):
            _row_copies(l, base, j0, v_hbm, buf, to_hbm=False)

        def col_body(c, carry2):
            off = c * _LANES
            xs = [buf[pl.ds(l * _NJ * _D + off, _LANES)] for l in range(_L)]
            for a, b2 in _NET:
                lo = jnp.minimum(xs[a], xs[b2])
                hi = jnp.maximum(xs[a], xs[b2])
                xs[a] = lo
                xs[b2] = hi
            for l in range(_L):
                buf[pl.ds(l * _NJ * _D + off, _LANES)] = xs[l]
            return carry2
        lax.fori_loop(0, _NJ * _D // _LANES, col_body, 0)

        for l in range(_L):
            _row_copies(l, base, j0, out_hbm, buf, to_hbm=True)
        return carry

    lax.fori_loop(0, _CPW, chunk_body, 0)


_sc_sort = pl.kernel(
    _sc_body,
    out_type=jax.ShapeDtypeStruct((_B * _S * _D,), jnp.float32),
    mesh=plsc.VectorSubcoreMesh(core_axis_name="c", subcore_axis_name="s"),
    scratch_types=[pltpu.VMEM((_L * _NJ * _D,), jnp.float32)],
)


def kernel(q, k, v):
    del q, k
    out = _sc_sort(v.reshape(_B * _S * _D))
    return out.reshape(_B, _S, _D)


# speedup vs baseline: 10.1872x; 2.4621x over previous
import jax
import jax.numpy as jnp
from jax import lax
from jax.experimental import pallas as pl
from jax.experimental.pallas import tpu as pltpu
from jax.experimental.pallas import tpu_sc as plsc

_B, _S, _D = 4, 4160, 1024
_L = 13
_G = _S // _L
_NP = 8
_DH = 512
_LANES = 16
_NW = 32
_UNITS = _B * (_G // _NP) * (_D // _DH)
_UPW = _UNITS // _NW
_CPD = _DH // _LANES

_NET = (
    (0, 1), (2, 3), (4, 5), (6, 7), (8, 9), (10, 11),
    (0, 2), (1, 3), (4, 6), (5, 7), (8, 10), (9, 11),
    (1, 2), (5, 6), (9, 10),
    (0, 4), (1, 5), (2, 6), (3, 7), (8, 12),
    (2, 4), (3, 5), (10, 12),
    (1, 2), (3, 4), (5, 6), (9, 10), (11, 12),
    (0, 8), (1, 9), (2, 10), (3, 11), (4, 12),
    (4, 8), (5, 9), (6, 10), (7, 11),
    (2, 4), (3, 5), (6, 8), (7, 9), (10, 12),
    (1, 2), (3, 4), (5, 6), (7, 8), (9, 10), (11, 12),
)


def _net_sorted(xs):
    xs = list(xs)
    for a, b in _NET:
        lo = jnp.minimum(xs[a], xs[b])
        hi = jnp.maximum(xs[a], xs[b])
        xs[a] = lo
        xs[b] = hi
    return xs


def _sort_window(buf, is_w0):
    @pl.when(jnp.logical_not(is_w0))
    def _():
        def body(i, carry):
            p = i >> 5
            off = (i & (_CPD - 1)) * _LANES
            ys = _net_sorted([buf[t, p, pl.ds(off, _LANES)]
                              for t in range(_L)])
            for k in range(_L):
                buf[k, p, pl.ds(off, _LANES)] = ys[k]
            return carry
        lax.fori_loop(0, _NP * _CPD, body, 0)

    @pl.when(is_w0)
    def _():
        def body_rot(i, carry):
            p = i >> 5
            off = (i & (_CPD - 1)) * _LANES
            ys = _net_sorted([buf[t, p, pl.ds(off, _LANES)]
                              for t in range(_L)])
            for k in range(_L):
                buf[(k + 1) % _L, p, pl.ds(off, _LANES)] = ys[k]
            return carry
        lax.fori_loop(0, (_NP - 1) * _CPD, body_rot, 0)

        def body_p7(c, carry):
            off = c * _LANES
            ys = _net_sorted([buf[t, _NP - 1, pl.ds(off, _LANES)]
                              for t in range(_L)])
            for k in range(_L):
                buf[k, _NP - 1, pl.ds(off, _LANES)] = ys[k]
            return carry
        lax.fori_loop(0, _CPD, body_p7, 0)


def _sc_body(v_hbm, out_hbm, buf):
    wid = lax.axis_index("s") * 2 + lax.axis_index("c")

    def unit_body(u, carry):
        g = wid * _UPW + u
        b = g // (_UNITS // _B)
        r = g % (_UNITS // _B)
        w = r >> 1
        r0 = w * _NP
        dc0 = (r & 1) * _DH
        src = v_hbm.at[b, :, pl.ds(r0, _NP), pl.ds(dc0, _DH)]
        pltpu.sync_copy(src, buf)
        _sort_window(buf, w == 0)
        dst = out_hbm.at[b, :, pl.ds(r0, _NP), pl.ds(dc0, _DH)]
        pltpu.sync_copy(buf, dst)
        return carry

    lax.fori_loop(0, _UPW, unit_body, 0)


_sc_sort = pl.kernel(
    _sc_body,
    out_type=jax.ShapeDtypeStruct((_B, _L, _G, _D), jnp.float32),
    mesh=plsc.VectorSubcoreMesh(core_axis_name="c", subcore_axis_name="s"),
    scratch_types=[pltpu.VMEM((_L, _NP, _DH), jnp.float32)],
)


def kernel(q, k, v):
    del q, k
    out = _sc_sort(v.reshape(_B, _L, _G, _D))
    return out.reshape(_B, _S, _D)


# speedup vs baseline: 11.3602x; 1.1151x over previous
import jax
import jax.numpy as jnp
from jax import lax
from jax.experimental import pallas as pl
from jax.experimental.pallas import tpu as pltpu
from jax.experimental.pallas import tpu_sc as plsc

_B, _S, _D = 4, 4160, 1024
_L = 13
_G = _S // _L
_NP = 8
_DH = 512
_LANES = 16
_NW = 32
_UNITS = _B * (_G // _NP) * (_D // _DH)
_UPW = _UNITS // _NW
_CPD = _DH // _LANES

_NET = (
    (0, 1), (2, 3), (4, 5), (6, 7), (8, 9), (10, 11),
    (0, 2), (1, 3), (4, 6), (5, 7), (8, 10), (9, 11),
    (1, 2), (5, 6), (9, 10),
    (0, 4), (1, 5), (2, 6), (3, 7), (8, 12),
    (2, 4), (3, 5), (10, 12),
    (1, 2), (3, 4), (5, 6), (9, 10), (11, 12),
    (0, 8), (1, 9), (2, 10), (3, 11), (4, 12),
    (4, 8), (5, 9), (6, 10), (7, 11),
    (2, 4), (3, 5), (6, 8), (7, 9), (10, 12),
    (1, 2), (3, 4), (5, 6), (7, 8), (9, 10), (11, 12),
)


def _net_sorted(xs):
    xs = list(xs)
    for a, b in _NET:
        lo = jnp.minimum(xs[a], xs[b])
        hi = jnp.maximum(xs[a], xs[b])
        xs[a] = lo
        xs[b] = hi
    return xs


def _sort_window(buf, is_w0):
    @pl.when(jnp.logical_not(is_w0))
    def _():
        def body(i, carry):
            p = i >> 5
            off = (i & (_CPD - 1)) * _LANES
            ys = _net_sorted([buf[t, p, pl.ds(off, _LANES)]
                              for t in range(_L)])
            for k in range(_L):
                buf[k, p, pl.ds(off, _LANES)] = ys[k]
            return carry
        lax.fori_loop(0, _NP * _CPD, body, 0)

    @pl.when(is_w0)
    def _():
        def body_rot(i, carry):
            p = i >> 5
            off = (i & (_CPD - 1)) * _LANES
            ys = _net_sorted([buf[t, p, pl.ds(off, _LANES)]
                              for t in range(_L)])
            for k in range(_L):
                buf[(k + 1) % _L, p, pl.ds(off, _LANES)] = ys[k]
            return carry
        lax.fori_loop(0, (_NP - 1) * _CPD, body_rot, 0)

        def body_p7(c, carry):
            off = c * _LANES
            ys = _net_sorted([buf[t, _NP - 1, pl.ds(off, _LANES)]
                              for t in range(_L)])
            for k in range(_L):
                buf[k, _NP - 1, pl.ds(off, _LANES)] = ys[k]
            return carry
        lax.fori_loop(0, _CPD, body_p7, 0)


def _sc_body(v_hbm, out_hbm, buf_a, buf_b, sa_i, sb_i, sa_o, sb_o):
    wid = lax.axis_index("s") * 2 + lax.axis_index("c")

    def unit(hbm, g):
        b = g // (_UNITS // _B)
        r = g % (_UNITS // _B)
        w = r >> 1
        r0 = w * _NP
        dc0 = (r & 1) * _DH
        return hbm.at[b, :, pl.ds(r0, _NP), pl.ds(dc0, _DH)], w == 0

    def pair_body(p, carry):
        ga = wid * _UPW + 2 * p
        gb = ga + 1
        src_a, w0_a = unit(v_hbm, ga)
        src_b, w0_b = unit(v_hbm, gb)
        ha = pltpu.async_copy(src_a, buf_a, sa_i)
        hb = pltpu.async_copy(src_b, buf_b, sb_i)
        ha.wait()
        _sort_window(buf_a, w0_a)
        dst_a, _ = unit(out_hbm, ga)
        oa = pltpu.async_copy(buf_a, dst_a, sa_o)
        hb.wait()
        _sort_window(buf_b, w0_b)
        dst_b, _ = unit(out_hbm, gb)
        ob = pltpu.async_copy(buf_b, dst_b, sb_o)
        oa.wait()
        ob.wait()
        return carry

    lax.fori_loop(0, _UPW // 2, pair_body, 0)


_sc_sort = pl.kernel(
    _sc_body,
    out_type=jax.ShapeDtypeStruct((_B, _L, _G, _D), jnp.float32),
    mesh=plsc.VectorSubcoreMesh(core_axis_name="c", subcore_axis_name="s"),
    scratch_types=[
        pltpu.VMEM((_L, _NP, _DH), jnp.float32),
        pltpu.VMEM((_L, _NP, _DH), jnp.float32),
        pltpu.SemaphoreType.DMA,
        pltpu.SemaphoreType.DMA,
        pltpu.SemaphoreType.DMA,
        pltpu.SemaphoreType.DMA,
    ],
)


def kernel(q, k, v):
    del q, k
    out = _sc_sort(v.reshape(_B, _L, _G, _D))
    return out.reshape(_B, _S, _D)


# speedup vs baseline: 11.6409x; 1.0247x over previous
import jax
import jax.numpy as jnp
from jax import lax
from jax.experimental import pallas as pl
from jax.experimental.pallas import tpu as pltpu
from jax.experimental.pallas import tpu_sc as plsc

_B, _S, _D = 4, 4160, 1024
_L = 13
_G = _S // _L
_NP = 8
_DH = 512
_LANES = 16
_NW = 32
_UNITS = _B * (_G // _NP) * (_D // _DH)
_UPW = _UNITS // _NW
_CPD = _DH // _LANES

_NET = (
    (0, 1), (2, 3), (4, 5), (6, 7), (8, 9), (10, 11),
    (0, 2), (1, 3), (4, 6), (5, 7), (8, 10), (9, 11),
    (1, 2), (5, 6), (9, 10),
    (0, 4), (1, 5), (2, 6), (3, 7), (8, 12),
    (2, 4), (3, 5), (10, 12),
    (1, 2), (3, 4), (5, 6), (9, 10), (11, 12),
    (0, 8), (1, 9), (2, 10), (3, 11), (4, 12),
    (4, 8), (5, 9), (6, 10), (7, 11),
    (2, 4), (3, 5), (6, 8), (7, 9), (10, 12),
    (1, 2), (3, 4), (5, 6), (7, 8), (9, 10), (11, 12),
)


def _net_sorted(xs):
    xs = list(xs)
    for a, b in _NET:
        lo = jnp.minimum(xs[a], xs[b])
        hi = jnp.maximum(xs[a], xs[b])
        xs[a] = lo
        xs[b] = hi
    return xs


def _sort_window(buf, is_w0):
    @pl.when(jnp.logical_not(is_w0))
    def _():
        @plsc.parallel_loop(0, _NP * _CPD, unroll=2)
        def _body(i):
            p = i >> 5
            off = (i & (_CPD - 1)) * _LANES
            ys = _net_sorted([buf[t, p, pl.ds(off, _LANES)]
                              for t in range(_L)])
            for k in range(_L):
                buf[k, p, pl.ds(off, _LANES)] = ys[k]

    @pl.when(is_w0)
    def _():
        @plsc.parallel_loop(0, (_NP - 1) * _CPD, unroll=2)
        def _body_rot(i):
            p = i >> 5
            off = (i & (_CPD - 1)) * _LANES
            ys = _net_sorted([buf[t, p, pl.ds(off, _LANES)]
                              for t in range(_L)])
            for k in range(_L):
                buf[(k + 1) % _L, p, pl.ds(off, _LANES)] = ys[k]

        @plsc.parallel_loop(0, _CPD, unroll=2)
        def _body_p7(c):
            off = c * _LANES
            ys = _net_sorted([buf[t, _NP - 1, pl.ds(off, _LANES)]
                              for t in range(_L)])
            for k in range(_L):
                buf[k, _NP - 1, pl.ds(off, _LANES)] = ys[k]


def _sc_body(v_hbm, out_hbm, buf_a, buf_b, sa_i, sb_i, sa_o, sb_o):
    wid = lax.axis_index("s") * 2 + lax.axis_index("c")

    def unit(hbm, g):
        b = g // (_UNITS // _B)
        r = g % (_UNITS // _B)
        w = r >> 1
        r0 = w * _NP
        dc0 = (r & 1) * _DH
        return hbm.at[b, :, pl.ds(r0, _NP), pl.ds(dc0, _DH)], w == 0

    def pair_body(p, carry):
        ga = wid * _UPW + 2 * p
        gb = ga + 1
        src_a, w0_a = unit(v_hbm, ga)
        src_b, w0_b = unit(v_hbm, gb)
        ha = pltpu.async_copy(src_a, buf_a, sa_i)
        hb = pltpu.async_copy(src_b, buf_b, sb_i)
        ha.wait()
        _sort_window(buf_a, w0_a)
        dst_a, _ = unit(out_hbm, ga)
        oa = pltpu.async_copy(buf_a, dst_a, sa_o)
        hb.wait()
        _sort_window(buf_b, w0_b)
        dst_b, _ = unit(out_hbm, gb)
        ob = pltpu.async_copy(buf_b, dst_b, sb_o)
        oa.wait()
        ob.wait()
        return carry

    lax.fori_loop(0, _UPW // 2, pair_body, 0)


_sc_sort = pl.kernel(
    _sc_body,
    out_type=jax.ShapeDtypeStruct((_B, _L, _G, _D), jnp.float32),
    mesh=plsc.VectorSubcoreMesh(core_axis_name="c", subcore_axis_name="s"),
    scratch_types=[
        pltpu.VMEM((_L, _NP, _DH), jnp.float32),
        pltpu.VMEM((_L, _NP, _DH), jnp.float32),
        pltpu.SemaphoreType.DMA,
        pltpu.SemaphoreType.DMA,
        pltpu.SemaphoreType.DMA,
        pltpu.SemaphoreType.DMA,
    ],
)


def kernel(q, k, v):
    del q, k
    out = _sc_sort(v.reshape(_B, _L, _G, _D))
    return out.reshape(_B, _S, _D)
